# baseline (speedup 1.0000x reference)
import functools

import jax
import jax.numpy as jnp
from jax import lax
from jax.experimental import pallas as pl
from jax.experimental.pallas import tpu as pltpu
from jax.experimental.pallas import tpu_sc as plsc

VOCAB = 1000
D = 128
B = 16384
L = 50
LP = 64
NW = 32
RW = B // NW
G = 8
NG = RW // G
NGP = NG // 2
PW = D // 2
NPC = PW // 16


def _build():
    mesh = plsc.VectorSubcoreMesh(core_axis_name="c", subcore_axis_name="s")

    @functools.partial(
        pl.kernel,
        mesh=mesh,
        out_type=jax.ShapeDtypeStruct((B * D,), jnp.float32),
        compiler_params=pltpu.CompilerParams(needs_layout_passes=False),
        scratch_types=[
            pltpu.VMEM((VOCAB * PW,), jnp.int32),
            pltpu.VMEM((G * LP,), jnp.int32),
            pltpu.VMEM((G * LP,), jnp.int32),
            pltpu.VMEM((G * D,), jnp.float32),
            pltpu.VMEM((G * D,), jnp.float32),
            pltpu.SemaphoreType.DMA,
            pltpu.SemaphoreType.DMA,
            pltpu.SemaphoreType.DMA,
            pltpu.SemaphoreType.DMA,
        ],
    )
    def run(cap_hbm, emb_hbm, out_hbm, table_v,
            idx0, idx1, outv0, outv1, si0, si1, so0, so1):
        idx_bufs = (idx0, idx1)
        out_bufs = (outv0, outv1)
        idx_sems = (si0, si1)
        out_sems = (so0, so1)

        wid = lax.axis_index("s") * 2 + lax.axis_index("c")
        base = wid * RW
        pltpu.sync_copy(emb_hbm, table_v)
        iota = lax.iota(jnp.int32, 16)
        dnums = lax.GatherDimensionNumbers(
            offset_dims=(), collapsed_slice_dims=(0,), start_index_map=(0,))
        scale = jnp.float32(1.0 / L)

        def idx_copy(g, b):
            return pltpu.make_async_copy(
                cap_hbm.at[pl.ds((base + g * G) * LP, G * LP)],
                idx_bufs[b], idx_sems[b])

        def out_copy(g, b):
            return pltpu.make_async_copy(
                out_bufs[b],
                out_hbm.at[pl.ds((base + g * G) * D, G * D)], out_sems[b])

        tviews = [table_v.at[pl.ds(c * 16, VOCAB * PW - c * 16)]
                  for c in range(NPC)]

        def row_body(r, ibuf, obuf):
            tokd = [ibuf[pl.ds(r * LP + k * 16, 16)] * PW for k in range(4)]
            accs = [jnp.zeros((32,), jnp.bfloat16) for _ in range(2 * NPC)]
            for l in range(L):
                base_l = lax.gather(
                    tokd[l // 16],
                    jnp.full((16, 1), l % 16, jnp.int32),
                    dnums, slice_sizes=(1,),
                    mode=lax.GatherScatterMode.PROMISE_IN_BOUNDS,
                )
                addr = base_l + iota
                for c in range(NPC):
                    pk = plsc.load_gather(tviews[c], [addr])
                    j = 2 * c + l % 2
                    accs[j] = accs[j] + plsc.bitcast(pk, jnp.bfloat16)
            for c in range(NPC):
                lo0, hi0 = plsc.unpack(
                    accs[2 * c], format=plsc.PackFormat.INTERLEAVED)
                lo1, hi1 = plsc.unpack(
                    accs[2 * c + 1], format=plsc.PackFormat.INTERLEAVED)
                obuf[pl.ds(r * D + 2 * c * 16, 16)] = (lo0 + lo1) * scale
                obuf[pl.ds(r * D + (2 * c + 1) * 16, 16)] = (hi0 + hi1) * scale

        idx_copy(0, 0).start()
        idx_copy(1, 1).start()

        def pair_body(gp, carry):
            for b in range(2):
                g = gp * 2 + b
                idx_copy(g, b).wait()

                @pl.when(gp >= 1)
                def _():
                    out_copy(g, b).wait()

                lax.fori_loop(
                    0, G,
                    lambda r, _: (row_body(r, idx_bufs[b], out_bufs[b]), 0)[1],
                    0)
                out_copy(g, b).start()
                idx_copy(g + 2, b).start()
            return carry

        lax.fori_loop(0, NGP, pair_body, 0)

        idx_copy(NG, 0).wait()
        idx_copy(NG + 1, 1).wait()
        out_copy(NG - 2, 0).wait()
        out_copy(NG - 1, 1).wait()

    return run


_run = _build()


def _pack_table(emb):
    ebf = emb.astype(jnp.bfloat16).reshape(VOCAB, NPC, 2, 16)
    u = lax.bitcast_convert_type(ebf, jnp.uint16).astype(jnp.uint32)
    pk = u[:, :, 0, :] | (u[:, :, 1, :] << 16)
    return lax.bitcast_convert_type(pk, jnp.int32).reshape(VOCAB * PW)


def kernel(captions, emb):
    cap = captions.astype(jnp.int32)
    cap_flat = jnp.pad(cap, ((0, 2 * G), (0, LP - L))).reshape(-1)
    out = _run(cap_flat, _pack_table(emb))
    return out.reshape(B, D)


# speedup vs baseline: 1.0588x; 1.0588x over previous
import functools

import jax
import jax.numpy as jnp
from jax import lax
from jax.experimental import pallas as pl
from jax.experimental.pallas import tpu as pltpu
from jax.experimental.pallas import tpu_sc as plsc

VOCAB = 1000
D = 128
B = 16384
L = 50
LP = 64
NW = 32
RW = B // NW
G = 16
NG = RW // G
NGP = NG // 2
PW = D // 2
NPC = PW // 16


def _build():
    mesh = plsc.VectorSubcoreMesh(core_axis_name="c", subcore_axis_name="s")

    @functools.partial(
        pl.kernel,
        mesh=mesh,
        out_type=jax.ShapeDtypeStruct((B * D,), jnp.float32),
        compiler_params=pltpu.CompilerParams(needs_layout_passes=False),
        scratch_types=[
            pltpu.VMEM((VOCAB * PW,), jnp.int32),
            pltpu.VMEM((G * LP,), jnp.int32),
            pltpu.VMEM((G * LP,), jnp.int32),
            pltpu.VMEM((G * D,), jnp.float32),
            pltpu.VMEM((G * D,), jnp.float32),
            pltpu.SemaphoreType.DMA,
            pltpu.SemaphoreType.DMA,
            pltpu.SemaphoreType.DMA,
            pltpu.SemaphoreType.DMA,
        ],
    )
    def run(cap_hbm, emb_hbm, out_hbm, table_v,
            idx0, idx1, outv0, outv1, si0, si1, so0, so1):
        idx_bufs = (idx0, idx1)
        out_bufs = (outv0, outv1)
        idx_sems = (si0, si1)
        out_sems = (so0, so1)

        wid = lax.axis_index("s") * 2 + lax.axis_index("c")
        base = wid * RW
        pltpu.sync_copy(emb_hbm, table_v)
        iota = lax.iota(jnp.int32, 16)
        dnums = lax.GatherDimensionNumbers(
            offset_dims=(), collapsed_slice_dims=(0,), start_index_map=(0,))
        scale = jnp.float32(1.0 / L)

        def idx_copy(g, b):
            return pltpu.make_async_copy(
                cap_hbm.at[pl.ds((base + g * G) * LP, G * LP)],
                idx_bufs[b], idx_sems[b])

        def out_copy(g, b):
            return pltpu.make_async_copy(
                out_bufs[b],
                out_hbm.at[pl.ds((base + g * G) * D, G * D)], out_sems[b])

        tviews = [table_v.at[pl.ds(c * 16, VOCAB * PW - c * 16)]
                  for c in range(NPC)]

        def row_body(r, ibuf, obuf):
            tokd = [ibuf[pl.ds(r * LP + k * 16, 16)] * PW for k in range(4)]
            accs = [jnp.zeros((16,), jnp.float32) for _ in range(2 * NPC)]
            for l in range(L):
                base_l = lax.gather(
                    tokd[l // 16],
                    jnp.full((16, 1), l % 16, jnp.int32),
                    dnums, slice_sizes=(1,),
                    mode=lax.GatherScatterMode.PROMISE_IN_BOUNDS,
                )
                addr = base_l + iota
                for c in range(NPC):
                    pk = plsc.load_gather(tviews[c], [addr])
                    lo, hi = plsc.unpack(
                        plsc.bitcast(pk, jnp.bfloat16),
                        format=plsc.PackFormat.INTERLEAVED)
                    accs[2 * c] = accs[2 * c] + lo
                    accs[2 * c + 1] = accs[2 * c + 1] + hi
            for j in range(2 * NPC):
                obuf[pl.ds(r * D + j * 16, 16)] = accs[j] * scale

        idx_copy(0, 0).start()
        idx_copy(1, 1).start()

        def pair_body(gp, carry):
            for b in range(2):
                g = gp * 2 + b
                idx_copy(g, b).wait()

                @pl.when(gp >= 1)
                def _():
                    out_copy(g, b).wait()

                lax.fori_loop(
                    0, G,
                    lambda r, _: (row_body(r, idx_bufs[b], out_bufs[b]), 0)[1],
                    0, unroll=2)
                out_copy(g, b).start()
                idx_copy(g + 2, b).start()
            return carry

        lax.fori_loop(0, NGP, pair_body, 0)

        idx_copy(NG, 0).wait()
        idx_copy(NG + 1, 1).wait()
        out_copy(NG - 2, 0).wait()
        out_copy(NG - 1, 1).wait()

    return run


_run = _build()


def _pack_table(emb):
    ebf = emb.astype(jnp.bfloat16).reshape(VOCAB, NPC, 2, 16)
    u = lax.bitcast_convert_type(ebf, jnp.uint16).astype(jnp.uint32)
    pk = u[:, :, 0, :] | (u[:, :, 1, :] << 16)
    return lax.bitcast_convert_type(pk, jnp.int32).reshape(VOCAB * PW)


def kernel(captions, emb):
    cap = captions.astype(jnp.int32)
    cap_flat = jnp.pad(cap, ((0, 2 * G), (0, LP - L))).reshape(-1)
    out = _run(cap_flat, _pack_table(emb))
    return out.reshape(B, D)


# speedup vs baseline: 1.0648x; 1.0057x over previous
import functools

import jax
import jax.numpy as jnp
from jax import lax
from jax.experimental import pallas as pl
from jax.experimental.pallas import tpu as pltpu
from jax.experimental.pallas import tpu_sc as plsc

VOCAB = 1000
D = 128
B = 16384
L = 50
LP = 64
NW = 32
RW = B // NW
G = 16
NG = RW // G
NGP = NG // 2
PW = D // 2
NPC = PW // 16


def _build():
    mesh = plsc.VectorSubcoreMesh(core_axis_name="c", subcore_axis_name="s")

    @functools.partial(
        pl.kernel,
        mesh=mesh,
        out_type=jax.ShapeDtypeStruct((B * D,), jnp.float32),
        compiler_params=pltpu.CompilerParams(needs_layout_passes=False),
        scratch_types=[
            pltpu.VMEM((VOCAB * PW,), jnp.int32),
            pltpu.VMEM((G * LP,), jnp.int32),
            pltpu.VMEM((G * LP,), jnp.int32),
            pltpu.VMEM((G * D,), jnp.float32),
            pltpu.VMEM((G * D,), jnp.float32),
            pltpu.SemaphoreType.DMA,
            pltpu.SemaphoreType.DMA,
            pltpu.SemaphoreType.DMA,
            pltpu.SemaphoreType.DMA,
        ],
    )
    def run(cap_hbm, emb_hbm, out_hbm, table_v,
            idx0, idx1, outv0, outv1, si0, si1, so0, so1):
        idx_bufs = (idx0, idx1)
        out_bufs = (outv0, outv1)
        idx_sems = (si0, si1)
        out_sems = (so0, so1)

        wid = lax.axis_index("s") * 2 + lax.axis_index("c")
        base = wid * RW
        pltpu.sync_copy(emb_hbm, table_v)
        iota = lax.iota(jnp.int32, 16)
        dnums = lax.GatherDimensionNumbers(
            offset_dims=(), collapsed_slice_dims=(0,), start_index_map=(0,))
        scale = jnp.float32(1.0 / L)

        def idx_copy(g, b):
            return pltpu.make_async_copy(
                cap_hbm.at[pl.ds((base + g * G) * LP, G * LP)],
                idx_bufs[b], idx_sems[b])

        def out_copy(g, b):
            return pltpu.make_async_copy(
                out_bufs[b],
                out_hbm.at[pl.ds((base + g * G) * D, G * D)], out_sems[b])

        tviews = [table_v.at[pl.ds(c * 16, VOCAB * PW - c * 16)]
                  for c in range(NPC)]

        def row_body(r, ibuf, obuf):
            accs = [jnp.zeros((16,), jnp.float32) for _ in range(2 * NPC)]
            for k in range(4):
                tv = ibuf[pl.ds(r * LP + k * 16, 16)] * PW
                for i in range(16 if k < 3 else L - 48):
                    tpw = tv[i]
                    for c in range(NPC):
                        pk = tviews[c][pl.ds(tpw, 16)]
                        lo, hi = plsc.unpack(
                            plsc.bitcast(pk, jnp.bfloat16),
                            format=plsc.PackFormat.INTERLEAVED)
                        accs[2 * c] = accs[2 * c] + lo
                        accs[2 * c + 1] = accs[2 * c + 1] + hi
            for j in range(2 * NPC):
                obuf[pl.ds(r * D + j * 16, 16)] = accs[j] * scale

        idx_copy(0, 0).start()
        idx_copy(1, 1).start()

        def pair_body(gp, carry):
            for b in range(2):
                g = gp * 2 + b
                idx_copy(g, b).wait()

                @pl.when(gp >= 1)
                def _():
                    out_copy(g, b).wait()

                lax.fori_loop(
                    0, G,
                    lambda r, _: (row_body(r, idx_bufs[b], out_bufs[b]), 0)[1],
                    0, unroll=2)
                out_copy(g, b).start()
                idx_copy(g + 2, b).start()
            return carry

        lax.fori_loop(0, NGP, pair_body, 0)

        idx_copy(NG, 0).wait()
        idx_copy(NG + 1, 1).wait()
        out_copy(NG - 2, 0).wait()
        out_copy(NG - 1, 1).wait()

    return run


_run = _build()


def _pack_table(emb):
    ebf = emb.astype(jnp.bfloat16).reshape(VOCAB, NPC, 2, 16)
    u = lax.bitcast_convert_type(ebf, jnp.uint16).astype(jnp.uint32)
    pk = u[:, :, 0, :] | (u[:, :, 1, :] << 16)
    return lax.bitcast_convert_type(pk, jnp.int32).reshape(VOCAB * PW)


def kernel(captions, emb):
    cap = captions.astype(jnp.int32)
    cap_flat = jnp.pad(cap, ((0, 2 * G), (0, LP - L))).reshape(-1)
    out = _run(cap_flat, _pack_table(emb))
    return out.reshape(B, D)
